# baseline (speedup 1.0000x reference)
import functools

import jax
import jax.numpy as jnp
from jax import lax
from jax.experimental import pallas as pl
from jax.experimental.pallas import tpu as pltpu
from jax.experimental.pallas import tpu_sc as plsc

E = 8
TOPK = 2
D = 768
F = 768
M = 2048
P = M * TOPK
T = 512
NT = P // T + E
R = NT * T

NW = 32
CHUNK = P // NW
NVEC = CHUNK // 16
MT = M // NW

_MESH = plsc.VectorSubcoreMesh(core_axis_name="c", subcore_axis_name="s")
_SC_PARAMS = pltpu.CompilerParams(needs_layout_passes=False)


def _wid():
    return lax.axis_index("s") * 2 + lax.axis_index("c")


@functools.partial(
    pl.kernel,
    out_type=jax.ShapeDtypeStruct((NW, 16), jnp.int32),
    mesh=_MESH,
    compiler_params=_SC_PARAMS,
    scratch_types=[
        pltpu.VMEM((CHUNK,), jnp.int32),
        pltpu.VMEM((16,), jnp.int32),
    ],
)
def _phase_a1(ids_hbm, cnt_hbm, ids_v, cnt_v):
    w = _wid()
    pltpu.sync_copy(ids_hbm.at[pl.ds(w * CHUNK, CHUNK)], ids_v)
    lanes = lax.iota(jnp.int32, 16)

    def vbody(vi, cnt):
        v = ids_v[pl.ds(vi * 16, 16)]

        def ebody(e, cnt):
            pc = plsc.all_reduce_population_count(v == e)
            return cnt + jnp.where(lanes == e, pc, 0)

        return lax.fori_loop(0, E, ebody, cnt)

    cnt_v[...] = lax.fori_loop(0, NVEC, vbody, jnp.zeros((16,), jnp.int32))
    pltpu.sync_copy(cnt_v, cnt_hbm.at[w])


@functools.partial(
    pl.kernel,
    out_type=(
        jax.ShapeDtypeStruct((P,), jnp.int32),
        jax.ShapeDtypeStruct((R, D), jnp.float32),
        jax.ShapeDtypeStruct((64,), jnp.int32),
    ),
    mesh=_MESH,
    compiler_params=_SC_PARAMS,
    scratch_types=[
        pltpu.VMEM((CHUNK,), jnp.int32),
        pltpu.VMEM((NW, 16), jnp.int32),
        pltpu.VMEM((CHUNK,), jnp.int32),
        pltpu.VMEM((CHUNK,), jnp.int32),
        pltpu.VMEM((CHUNK, D), jnp.float32),
        pltpu.VMEM((64,), jnp.int32),
        pltpu.SemaphoreType.DMA,
        pltpu.SemaphoreType.DMA,
    ],
)
def _phase_a2(ids_hbm, cnt_hbm, hid_hbm,
              pos_hbm, xg_hbm, meta_hbm,
              ids_v, allcnt_v, rowbuf_v, tok_v,
              xrows_v, meta_v, gsem, ssem):
    w = _wid()
    base = w * CHUNK
    lanes = lax.iota(jnp.int32, 16)
    zero16 = jnp.zeros((16,), jnp.int32)

    def tbody(vi, _):
        jvec = base + vi * 16 + lanes
        tok_v[pl.ds(vi * 16, 16)] = jvec >> 1
        return _

    lax.fori_loop(0, NVEC, tbody, 0)
    gcp = pltpu.async_copy(hid_hbm.at[tok_v], xrows_v, gsem)
    cp_ids = pltpu.async_copy(ids_hbm.at[pl.ds(base, CHUNK)], ids_v, ssem)
    cp_cnt = pltpu.async_copy(cnt_hbm, allcnt_v, ssem)
    cp_ids.wait()
    cp_cnt.wait()

    def accbody(i, carry):
        pre, tot = carry
        row = allcnt_v[i, :]
        pre = pre + jnp.where(i < w, row, 0)
        return pre, tot + row

    pre, tot = lax.fori_loop(0, NW, accbody, (zero16, zero16))

    ntiles = (tot + (T - 1)) >> 9
    incl = plsc.cumsum(ntiles)
    tstart = incl - ntiles
    mybase = tstart * T + pre

    ntot = jnp.sum(ntiles)

    @pl.when(w == 0)
    def _():
        def half_body(half, _):
            ivec = lanes + half * 16

            def ebody(e, acc):
                ts_e = jnp.sum(jnp.where(lanes == e, tstart, 0))
                return acc + jnp.where(ivec >= ts_e, 1, 0)

            cntv = lax.fori_loop(0, E, ebody, zero16)
            meta_v[pl.ds(half * 16, 16)] = cntv - 1
            meta_v[pl.ds(32 + half * 16, 16)] = jnp.where(ivec < ntot, 1, 0)
            return _

        lax.fori_loop(0, 2, half_body, 0)
        tailv = meta_v[pl.ds(48, 16)]
        meta_v[pl.ds(48, 16)] = jnp.where(lanes == 15, ntot, tailv)
        pltpu.sync_copy(meta_v, meta_hbm)

    def vbody(vi, curbase):
        v = ids_v[pl.ds(vi * 16, 16)]

        def ebody(e, ec):
            curbase, posv = ec
            m = v == e
            mi = jnp.where(m, 1, 0)
            excl = plsc.cumsum(mi) - mi
            be = jnp.sum(jnp.where(lanes == e, curbase, 0))
            posv = jnp.where(m, be + excl, posv)
            curbase = curbase + jnp.where(
                lanes == e, plsc.all_reduce_population_count(m), 0)
            return curbase, posv

        curbase, posv = lax.fori_loop(0, E, ebody, (curbase, zero16))
        rowbuf_v[pl.ds(vi * 16, 16)] = posv
        return curbase

    lax.fori_loop(0, NVEC, vbody, mybase)

    gcp.wait()
    cp1 = pltpu.async_copy(rowbuf_v, pos_hbm.at[pl.ds(base, CHUNK)], ssem)
    cp2 = pltpu.async_copy(xrows_v, xg_hbm.at[rowbuf_v], ssem)
    cp1.wait()
    cp2.wait()


def _phase_b_body(meta_ref, xg_ref, w1_ref, w2_ref, y_ref):
    i = pl.program_id(0)

    @pl.when(meta_ref[32 + i] == 1)
    def _():
        x = xg_ref[...]
        h = lax.dot_general(x, w1_ref[0], (((1,), (1,)), ((), ())),
                            preferred_element_type=jnp.float32)
        gate = h[:, :F]
        up = h[:, F:]
        act = gate * jax.nn.sigmoid(gate) * up
        y_ref[...] = lax.dot_general(act, w2_ref[0], (((1,), (1,)), ((), ())),
                                     preferred_element_type=jnp.float32)


def _phase_b(meta, xg, w1, w2):
    def live(i, m):
        return jnp.minimum(i, m[63] - 1)

    grid_spec = pltpu.PrefetchScalarGridSpec(
        num_scalar_prefetch=1,
        grid=(NT,),
        in_specs=[
            pl.BlockSpec((T, D), lambda i, m: (live(i, m), 0)),
            pl.BlockSpec((1, 2 * F, D), lambda i, m: (m[live(i, m)], 0, 0)),
            pl.BlockSpec((1, D, F), lambda i, m: (m[live(i, m)], 0, 0)),
        ],
        out_specs=pl.BlockSpec((T, D), lambda i, m: (live(i, m), 0)),
    )
    return pl.pallas_call(
        _phase_b_body,
        grid_spec=grid_spec,
        out_shape=jax.ShapeDtypeStruct((R, D), jnp.float32),
        compiler_params=pltpu.CompilerParams(
            vmem_limit_bytes=100 * 1024 * 1024),
    )(meta, xg, w1, w2)


@functools.partial(
    pl.kernel,
    out_type=jax.ShapeDtypeStruct((M, D), jnp.float32),
    mesh=_MESH,
    compiler_params=_SC_PARAMS,
    scratch_types=[
        pltpu.VMEM((2 * MT,), jnp.int32),
        pltpu.VMEM((2 * MT,), jnp.float32),
        pltpu.VMEM((MT,), jnp.int32),
        pltpu.VMEM((MT,), jnp.int32),
        pltpu.VMEM((MT,), jnp.float32),
        pltpu.VMEM((MT,), jnp.float32),
        pltpu.VMEM((2, 16, D), jnp.float32),
        pltpu.VMEM((2, 16, D), jnp.float32),
        pltpu.VMEM((MT, D), jnp.float32),
        pltpu.SemaphoreType.DMA,
        pltpu.SemaphoreType.DMA,
        pltpu.SemaphoreType.DMA,
    ],
)
def _phase_c(ybuf_hbm, pos_hbm, tw_hbm, out_hbm,
             pch_v, twch_v, idx0_v, idx1_v, w0_v, w1_v, r0_v, r1_v, out_v,
             sem0, sem1, osem):
    w = _wid()
    t0 = w * MT
    NCH = MT // 16
    cpa = pltpu.async_copy(pos_hbm.at[pl.ds(2 * t0, 2 * MT)], pch_v, sem0)
    cpb = pltpu.async_copy(tw_hbm.at[pl.ds(2 * t0, 2 * MT)], twch_v, sem1)
    cpa.wait()
    cpb.wait()
    lanes = lax.iota(jnp.int32, 16)
    for g in range(NCH):
        ev = lanes * 2 + g * 32
        od = ev + 1
        idx0_v[pl.ds(g * 16, 16)] = plsc.load_gather(pch_v, [ev])
        idx1_v[pl.ds(g * 16, 16)] = plsc.load_gather(pch_v, [od])
        w0_v[pl.ds(g * 16, 16)] = plsc.load_gather(twch_v, [ev])
        w1_v[pl.ds(g * 16, 16)] = plsc.load_gather(twch_v, [od])

    def issue(k):
        b = k % 2
        c0 = pltpu.async_copy(
            ybuf_hbm.at[idx0_v.at[pl.ds(k * 16, 16)]], r0_v.at[b], sem0)
        c1 = pltpu.async_copy(
            ybuf_hbm.at[idx1_v.at[pl.ds(k * 16, 16)]], r1_v.at[b], sem1)
        return c0, c1

    pend = {0: issue(0), 1: issue(1)}
    ocps = []
    for k in range(NCH):
        b = k % 2
        c0, c1 = pend.pop(k)
        c0.wait()
        c1.wait()

        def rbody(r, _):
            rv = jnp.full((16,), k * 16 + r, jnp.int32)
            w0b = plsc.load_gather(w0_v, [rv])
            w1b = plsc.load_gather(w1_v, [rv])
            for cc in range(D // 16):
                sl = pl.ds(cc * 16, 16)
                out_v[k * 16 + r, sl] = r0_v[b, r, sl] * w0b + r1_v[b, r, sl] * w1b
            return _

        lax.fori_loop(0, 16, rbody, 0)
        if k + 2 < NCH:
            pend[k + 2] = issue(k + 2)
        ocps.append(pltpu.async_copy(
            out_v.at[pl.ds(k * 16, 16)], out_hbm.at[pl.ds(t0 + k * 16, 16)],
            osem))
    for c in ocps:
        c.wait()


def kernel(hidden_states, w1, w2, topk_weights, topk_ids):
    ids = topk_ids.astype(jnp.int32).reshape(P)
    tw = topk_weights.astype(jnp.float32).reshape(P)
    cnt = _phase_a1(ids)
    pos, xg, meta = _phase_a2(ids, cnt, hidden_states)
    ybuf = _phase_b(meta, xg, w1, w2)
    return _phase_c(ybuf, pos, tw)


# speedup vs baseline: 1.1563x; 1.1563x over previous
import functools

import jax
import jax.numpy as jnp
from jax import lax
from jax.experimental import pallas as pl
from jax.experimental.pallas import tpu as pltpu
from jax.experimental.pallas import tpu_sc as plsc

E = 8
TOPK = 2
D = 768
F = 768
M = 2048
P = M * TOPK
T = 512
NT = P // T + E
R = NT * T

NW = 32
CHUNK = P // NW
NVEC = CHUNK // 16
MT = M // NW
D2 = D // 2

_MESH = plsc.VectorSubcoreMesh(core_axis_name="c", subcore_axis_name="s")
_SC_PARAMS = pltpu.CompilerParams(needs_layout_passes=False)


def _wid():
    return lax.axis_index("s") * 2 + lax.axis_index("c")


@functools.partial(
    pl.kernel,
    out_type=jax.ShapeDtypeStruct((NW, 16), jnp.int32),
    mesh=_MESH,
    compiler_params=_SC_PARAMS,
    scratch_types=[
        pltpu.VMEM((CHUNK,), jnp.int32),
        pltpu.VMEM((16,), jnp.int32),
    ],
)
def _phase_a1(ids_hbm, cnt_hbm, ids_v, cnt_v):
    w = _wid()
    pltpu.sync_copy(ids_hbm.at[pl.ds(w * CHUNK, CHUNK)], ids_v)
    lanes = lax.iota(jnp.int32, 16)

    def vbody(vi, cnt):
        v = ids_v[pl.ds(vi * 16, 16)]

        def ebody(e, cnt):
            pc = plsc.all_reduce_population_count(v == e)
            return cnt + jnp.where(lanes == e, pc, 0)

        return lax.fori_loop(0, E, ebody, cnt)

    cnt_v[...] = lax.fori_loop(0, NVEC, vbody, jnp.zeros((16,), jnp.int32))
    pltpu.sync_copy(cnt_v, cnt_hbm.at[w])


@functools.partial(
    pl.kernel,
    out_type=(
        jax.ShapeDtypeStruct((P,), jnp.int32),
        jax.ShapeDtypeStruct((R, D2), jnp.int32),
        jax.ShapeDtypeStruct((64,), jnp.int32),
    ),
    mesh=_MESH,
    compiler_params=_SC_PARAMS,
    scratch_types=[
        pltpu.VMEM((CHUNK,), jnp.int32),
        pltpu.VMEM((NW, 16), jnp.int32),
        pltpu.VMEM((CHUNK,), jnp.int32),
        pltpu.VMEM((CHUNK,), jnp.int32),
        pltpu.VMEM((CHUNK, D2), jnp.int32),
        pltpu.VMEM((64,), jnp.int32),
        pltpu.SemaphoreType.DMA,
        pltpu.SemaphoreType.DMA,
    ],
)
def _phase_a2(ids_hbm, cnt_hbm, hid_hbm,
              pos_hbm, xg_hbm, meta_hbm,
              ids_v, allcnt_v, rowbuf_v, tok_v,
              xrows_v, meta_v, gsem, ssem):
    w = _wid()
    base = w * CHUNK
    lanes = lax.iota(jnp.int32, 16)
    zero16 = jnp.zeros((16,), jnp.int32)

    def tbody(vi, _):
        jvec = base + vi * 16 + lanes
        tok_v[pl.ds(vi * 16, 16)] = jvec >> 1
        return _

    lax.fori_loop(0, NVEC, tbody, 0)
    gcp = pltpu.async_copy(hid_hbm.at[tok_v], xrows_v, gsem)
    cp_ids = pltpu.async_copy(ids_hbm.at[pl.ds(base, CHUNK)], ids_v, ssem)
    cp_cnt = pltpu.async_copy(cnt_hbm, allcnt_v, ssem)
    cp_ids.wait()
    cp_cnt.wait()

    def accbody(i, carry):
        pre, tot = carry
        row = allcnt_v[i, :]
        pre = pre + jnp.where(i < w, row, 0)
        return pre, tot + row

    pre, tot = lax.fori_loop(0, NW, accbody, (zero16, zero16))

    ntiles = (tot + (T - 1)) >> 9
    incl = plsc.cumsum(ntiles)
    tstart = incl - ntiles
    mybase = tstart * T + pre

    ntot = jnp.sum(ntiles)

    @pl.when(w == 0)
    def _():
        def half_body(half, _):
            ivec = lanes + half * 16

            def ebody(e, acc):
                ts_e = jnp.sum(jnp.where(lanes == e, tstart, 0))
                return acc + jnp.where(ivec >= ts_e, 1, 0)

            cntv = lax.fori_loop(0, E, ebody, zero16)
            meta_v[pl.ds(half * 16, 16)] = cntv - 1
            meta_v[pl.ds(32 + half * 16, 16)] = jnp.where(ivec < ntot, 1, 0)
            return _

        lax.fori_loop(0, 2, half_body, 0)
        tailv = meta_v[pl.ds(48, 16)]
        meta_v[pl.ds(48, 16)] = jnp.where(lanes == 15, ntot, tailv)
        pltpu.sync_copy(meta_v, meta_hbm)

    def vbody(vi, curbase):
        v = ids_v[pl.ds(vi * 16, 16)]

        def ebody(e, ec):
            curbase, posv = ec
            m = v == e
            mi = jnp.where(m, 1, 0)
            excl = plsc.cumsum(mi) - mi
            be = jnp.sum(jnp.where(lanes == e, curbase, 0))
            posv = jnp.where(m, be + excl, posv)
            curbase = curbase + jnp.where(
                lanes == e, plsc.all_reduce_population_count(m), 0)
            return curbase, posv

        curbase, posv = lax.fori_loop(0, E, ebody, (curbase, zero16))
        rowbuf_v[pl.ds(vi * 16, 16)] = posv
        return curbase

    lax.fori_loop(0, NVEC, vbody, mybase)

    gcp.wait()
    cp1 = pltpu.async_copy(rowbuf_v, pos_hbm.at[pl.ds(base, CHUNK)], ssem)
    cp2 = pltpu.async_copy(xrows_v, xg_hbm.at[rowbuf_v], ssem)
    cp1.wait()
    cp2.wait()


def _phase_b_body(meta_ref, xg_ref, w1_ref, w2_ref, y_ref):
    i = pl.program_id(0)

    @pl.when(meta_ref[32 + i] == 1)
    def _():
        xp = xg_ref[...]
        lo = lax.bitcast_convert_type(xp << 16, jnp.float32)
        hi = lax.bitcast_convert_type(
            xp & jnp.int32(-65536), jnp.float32)
        x = jnp.concatenate([lo, hi], axis=1)
        h = lax.dot_general(x, w1_ref[0], (((1,), (1,)), ((), ())),
                            preferred_element_type=jnp.float32)
        gate = h[:, :F]
        up = h[:, F:]
        act = gate * jax.nn.sigmoid(gate) * up
        y_ref[...] = lax.dot_general(act, w2_ref[0], (((1,), (1,)), ((), ())),
                                     preferred_element_type=jnp.float32)


def _phase_b(meta, xg, w1, w2):
    def live(i, m):
        return jnp.minimum(i, m[63] - 1)

    grid_spec = pltpu.PrefetchScalarGridSpec(
        num_scalar_prefetch=1,
        grid=(NT,),
        in_specs=[
            pl.BlockSpec((T, D2), lambda i, m: (live(i, m), 0)),
            pl.BlockSpec((1, 2 * F, D), lambda i, m: (m[live(i, m)], 0, 0)),
            pl.BlockSpec((1, D, F), lambda i, m: (m[live(i, m)], 0, 0)),
        ],
        out_specs=pl.BlockSpec((T, D), lambda i, m: (live(i, m), 0)),
    )
    return pl.pallas_call(
        _phase_b_body,
        grid_spec=grid_spec,
        out_shape=jax.ShapeDtypeStruct((R, D), jnp.float32),
        compiler_params=pltpu.CompilerParams(
            vmem_limit_bytes=100 * 1024 * 1024),
    )(meta, xg, w1, w2)


@functools.partial(
    pl.kernel,
    out_type=jax.ShapeDtypeStruct((M, D), jnp.float32),
    mesh=_MESH,
    compiler_params=_SC_PARAMS,
    scratch_types=[
        pltpu.VMEM((2 * MT,), jnp.int32),
        pltpu.VMEM((2 * MT,), jnp.float32),
        pltpu.VMEM((MT,), jnp.int32),
        pltpu.VMEM((MT,), jnp.int32),
        pltpu.VMEM((MT,), jnp.float32),
        pltpu.VMEM((MT,), jnp.float32),
        pltpu.VMEM((MT, D), jnp.float32),
        pltpu.VMEM((MT, D), jnp.float32),
        pltpu.SemaphoreType.DMA,
        pltpu.SemaphoreType.DMA,
    ],
)
def _phase_c(ybuf_hbm, pos_hbm, tw_hbm, out_hbm,
             pch_v, twch_v, idx0_v, idx1_v, w0_v, w1_v, acc_v, buf_v,
             sem0, sem1):
    w = _wid()
    t0 = w * MT
    cpa = pltpu.async_copy(pos_hbm.at[pl.ds(2 * t0, 2 * MT)], pch_v, sem0)
    cpb = pltpu.async_copy(tw_hbm.at[pl.ds(2 * t0, 2 * MT)], twch_v, sem1)
    cpa.wait()
    cpb.wait()
    lanes = lax.iota(jnp.int32, 16)
    for g in range(MT // 16):
        ev = lanes * 2 + g * 32
        od = ev + 1
        idx0_v[pl.ds(g * 16, 16)] = plsc.load_gather(pch_v, [ev])
        idx1_v[pl.ds(g * 16, 16)] = plsc.load_gather(pch_v, [od])
        w0_v[pl.ds(g * 16, 16)] = plsc.load_gather(twch_v, [ev])
        w1_v[pl.ds(g * 16, 16)] = plsc.load_gather(twch_v, [od])
    cp0 = pltpu.async_copy(ybuf_hbm.at[idx0_v], acc_v, sem0)
    cp1 = pltpu.async_copy(ybuf_hbm.at[idx1_v], buf_v, sem1)
    cp0.wait()
    cp1.wait()

    def rbody(r, _):
        rv = jnp.full((16,), r, jnp.int32)
        w0b = plsc.load_gather(w0_v, [rv])
        w1b = plsc.load_gather(w1_v, [rv])
        for cc in range(D // 16):
            sl = pl.ds(cc * 16, 16)
            acc_v[r, sl] = acc_v[r, sl] * w0b + buf_v[r, sl] * w1b
        return _

    lax.fori_loop(0, MT, rbody, 0)
    pltpu.sync_copy(acc_v, out_hbm.at[pl.ds(t0, MT)])


def kernel(hidden_states, w1, w2, topk_weights, topk_ids):
    ids = topk_ids.astype(jnp.int32).reshape(P)
    tw = topk_weights.astype(jnp.float32).reshape(P)
    cnt = _phase_a1(ids)
    hb = hidden_states.astype(jnp.bfloat16)
    hid_packed = lax.bitcast_convert_type(
        jnp.stack([hb[:, :D2], hb[:, D2:]], axis=-1), jnp.int32)
    pos, xg, meta = _phase_a2(ids, cnt, hid_packed)
    ybuf = _phase_b(meta, xg, w1, w2)
    return _phase_c(ybuf, pos, tw)
